# Initial kernel scaffold; baseline (speedup 1.0000x reference)
import jax
import jax.numpy as jnp
from jax.experimental import pallas as pl


def kernel(node_x, embedding, node_U1, node_U2, fc_w, fc_b):
    raise NotImplementedError("write your pallas kernel here")


# speedup vs baseline: 20.3485x; 20.3485x over previous
import functools

import numpy as np
import jax
import jax.numpy as jnp
from jax.experimental import pallas as pl
from jax.experimental.pallas import tpu as pltpu

_N_HEADS = 8
_SQRT_L = np.sqrt(np.float32(24.0), dtype=np.float32)
_BF = jnp.bfloat16


def _softmax_rows(x):
    m = jnp.max(x, axis=-1, keepdims=True)
    e = jnp.exp(x - m)
    return e / jnp.sum(e, axis=-1, keepdims=True)


def _bf16_dot(a, b, dims):
    return jax.lax.dot_general(a.astype(_BF), b.astype(_BF), (dims, ((), ())),
                               preferred_element_type=jnp.float32)


def _geanet_body(x_ref, xs_ref, emb_ref, u1_ref, u2_ref, out_l1, out_s,
                 *, l, N, d, k):
    x = x_ref[0]
    u1 = u1_ref[...]
    u2 = u2_ref[...]
    H = _N_HEADS
    unit = d // H
    A = N // H

    s_cols = []
    for t in range(l):
        p = jax.lax.dot_general(x[t].astype(_BF), u1, (((1,), (0,)), ((), ())),
                                preferred_element_type=jnp.float32)
        m = jnp.max(p, axis=0, keepdims=True)
        e = jnp.exp(p - m)
        z = e
        n = N
        while n > 8:
            n //= 2
            z = z[:n] + z[n:2 * n]
        z = jnp.sum(z, axis=0, keepdims=True)
        attn = e / z
        no = jax.lax.dot_general(attn.astype(_BF), u2, (((1,), (0,)), ((), ())),
                                 preferred_element_type=jnp.float32)
        nr = no.reshape(A, H, d)
        t1 = nr[:, 0:4] + nr[:, 4:8]
        t2 = t1[:, 0:2] + t1[:, 2:4]
        c = t2[:, 0] + t2[:, 1]
        w = d
        while w > H:
            w //= 2
            c = c[:, :w] + c[:, w:2 * w]
        g = c
        s_cols.append(jnp.concatenate([g[:, h:h + 1] for h in range(H)],
                                      axis=0))
    s_mat = jnp.concatenate(s_cols, axis=1)

    l1 = jax.lax.dot_general(xs_ref[0], emb_ref[...],
                             (((1,), (0,)), ((), ())),
                             preferred_element_type=jnp.float32)
    out_l1[0] = l1
    out_s[0] = s_mat


def _mask_body(adj_ref, out_ref, *, N, k):
    adj = adj_ref[0]
    bits = jax.lax.bitcast_convert_type(adj, jnp.int32)
    q_rank = N - k + 1

    lo = jnp.zeros((N, 1), jnp.int32)
    hi = jnp.full((N, 1), 0x7f800000, jnp.int32)

    def find_thresh(_, carry):
        lo, hi = carry
        mid = lo + ((hi - lo) >> 1)
        cnt = jnp.sum((bits <= mid).astype(jnp.int32), axis=1, keepdims=True)
        pred = cnt >= q_rank
        return (jnp.where(pred, lo, mid + 1), jnp.where(pred, mid, hi))

    lo, hi = jax.lax.fori_loop(0, 31, find_thresh, (lo, hi))
    thr = hi

    gt = bits > thr
    eq = bits == thr
    c_gt = jnp.sum(gt.astype(jnp.int32), axis=1, keepdims=True)
    lanes = jax.lax.broadcasted_iota(jnp.int32, (N, N), 1)

    lo2 = jnp.zeros((N, 1), jnp.int32)
    hi2 = jnp.full((N, 1), N, jnp.int32)

    def find_tie_cut(_, carry):
        lo2, hi2 = carry
        mid = lo2 + ((hi2 - lo2) >> 1)
        cnt = c_gt + jnp.sum((eq & (lanes < mid)).astype(jnp.int32),
                             axis=1, keepdims=True)
        pred = cnt >= k
        return (jnp.where(pred, lo2, mid + 1), jnp.where(pred, mid, hi2))

    lo2, hi2 = jax.lax.fori_loop(0, 10, find_tie_cut, (lo2, hi2))

    keep = gt | (eq & (lanes < hi2))
    out_ref[0] = adj * keep.astype(jnp.float32)


def kernel(node_x, embedding, node_U1, node_U2, fc_w, fc_b):
    b, l, N, d = node_x.shape
    H = _N_HEADS
    unit = d // H
    S = node_U1.shape[1]
    k = int(N * 0.8)

    u1bd = jnp.kron(jnp.eye(H, dtype=node_U1.dtype), node_U1).astype(_BF)
    u2bd = (node_U2[None, :, :, None]
            * jnp.eye(H, dtype=node_U2.dtype)[:, None, None, :]
            ).reshape(H * S, unit * H).astype(_BF)

    xsb = jnp.sum(node_x, axis=1).astype(_BF)

    body = functools.partial(_geanet_body, l=l, N=N, d=d, k=k)
    l1, s_mat = pl.pallas_call(
        body,
        grid=(b,),
        in_specs=[
            pl.BlockSpec((1, l, N, d), lambda i: (i, 0, 0, 0)),
            pl.BlockSpec((1, N, d), lambda i: (i, 0, 0)),
            pl.BlockSpec((d, N), lambda i: (0, 0)),
            pl.BlockSpec((d, H * S), lambda i: (0, 0)),
            pl.BlockSpec((H * S, d), lambda i: (0, 0)),
        ],
        out_specs=[
            pl.BlockSpec((1, N, N), lambda i: (i, 0, 0)),
            pl.BlockSpec((1, N, l), lambda i: (i, 0, 0)),
        ],
        out_shape=[
            jax.ShapeDtypeStruct((b, N, N), jnp.float32),
            jax.ShapeDtypeStruct((b, N, l), jnp.float32),
        ],
        compiler_params=pltpu.CompilerParams(
            dimension_semantics=("arbitrary",),
        ),
    )(node_x, xsb, embedding.T.astype(_BF), u1bd, u2bd)

    s = jnp.swapaxes(s_mat, 1, 2)
    adj_dyn_1 = jax.nn.softmax(
        jax.nn.relu(l1 / jnp.sqrt(jnp.float32(l))), axis=-1)
    adj_dyn_2 = jax.nn.softmax(
        jax.nn.relu(jnp.einsum('bcn,bcm->bnm', s, s) / jnp.sqrt(jnp.float32(l))),
        axis=-1)
    adj_f = jnp.concatenate([adj_dyn_1[..., None], adj_dyn_2[..., None]], axis=-1)
    adj_f = jnp.squeeze(jnp.matmul(adj_f, fc_w.T) + fc_b, axis=-1)
    adj = jax.nn.softmax(adj_f, axis=-1)

    mask_body = functools.partial(_mask_body, N=N, k=k)
    return pl.pallas_call(
        mask_body,
        grid=(b,),
        in_specs=[pl.BlockSpec((1, N, N), lambda i: (i, 0, 0))],
        out_specs=pl.BlockSpec((1, N, N), lambda i: (i, 0, 0)),
        out_shape=jax.ShapeDtypeStruct((b, N, N), jnp.float32),
        compiler_params=pltpu.CompilerParams(
            dimension_semantics=("arbitrary",),
        ),
    )(adj)
